# baseline (speedup 1.0000x reference)
import functools

import jax
import jax.numpy as jnp
from jax import lax
from jax.experimental import pallas as pl
from jax.experimental.pallas import tpu as pltpu
from jax.experimental.pallas import tpu_sc as plsc

_N = 10000
_F = 128
_E = 320000
_NC = 2
_NS = 16
_NW = _NC * _NS
_CH = 128
_CPW = 80
_EPAD = _NW * _CPW * _CH
_RPS = 640
_NBUF = 2
_WCH = 4
_HROWS = _N + 8
_CPW_A = 112
_CPW_B = 48


def _sc_aggregate(h_pad, src2, dst2):
    mesh = plsc.VectorSubcoreMesh(core_axis_name="c", subcore_axis_name="s")
    out_t = (jax.ShapeDtypeStruct((_N, _F), jnp.float32),
             jax.ShapeDtypeStruct((_N, _F), jnp.float32))

    @functools.partial(
        pl.kernel, mesh=mesh, out_type=out_t,
        scratch_types=[
            pltpu.VMEM((2, _WCH, _CH), jnp.int32),
            pltpu.VMEM((max(_CPW_A, _CPW_B), _CH), jnp.int32),
            pltpu.VMEM((_NBUF, _CH, _F), jnp.float32),
            pltpu.VMEM_SHARED((_N, _F), jnp.float32),
        ] + [pltpu.SemaphoreType.DMA] * 7)
    def k(h_hbm, src_hbm, dst_hbm, o0, o1,
          src_v, dst_v, rows_v, acc, *sems):
        gsem = sems[0:2]
        ssem = sems[2:4]
        isem = sems[4:6]
        dsem = sems[6]
        c = lax.axis_index("c")
        s = lax.axis_index("s")
        base = s * _RPS

        zvec = jnp.zeros((16,), jnp.float32)

        @pl.loop(0, _CH)
        def _(i):
            @pl.loop(0, _F, step=16)
            def _(j):
                rows_v[0, i, pl.ds(j, 16)] = zvec

        @pl.loop(0, 3)
        def _(kk):
            pltpu.sync_copy(rows_v.at[0],
                            acc.at[pl.ds(base + kk * _CH, _CH)])

        @pl.when(s < _NS - 1)
        def _():
            pltpu.sync_copy(rows_v.at[0],
                            acc.at[pl.ds(base + 3 * _CH, _CH)])
            pltpu.sync_copy(rows_v.at[0],
                            acc.at[pl.ds(base + 4 * _CH, _CH)])

        @pl.when(s == _NS - 1)
        def _():
            pltpu.sync_copy(rows_v.at[0, pl.ds(0, 16)],
                            acc.at[pl.ds(base + 3 * _CH, 16)])

        def copy_slab(src_ref, dst_ref):
            @pl.when(s < _NS - 1)
            def _():
                pltpu.sync_copy(src_ref.at[pl.ds(base, _RPS)],
                                dst_ref.at[pl.ds(base, _RPS)])

            @pl.when(s == _NS - 1)
            def _():
                pltpu.sync_copy(src_ref.at[pl.ds(_RPS * (_NS - 1), 400)],
                                dst_ref.at[pl.ds(_RPS * (_NS - 1), 400)])

        def load_idx(ch0, win, p):
            pltpu.async_copy(
                src_hbm.at[pl.ds(ch0 + win * _WCH, _WCH)],
                src_v.at[p], isem[p])

        def wait_idx(p):
            pltpu.make_async_copy(
                src_hbm.at[pl.ds(0, _WCH)], src_v.at[p],
                isem[p]).wait()

        def start_gather(win_p, row, b):
            pltpu.async_copy(h_hbm.at[src_v.at[win_p, row]],
                             rows_v.at[b], gsem[b])

        def wait_gather(win_p, row, b):
            pltpu.make_async_copy(h_hbm.at[src_v.at[win_p, row]],
                                  rows_v.at[b], gsem[b]).wait()

        def run_pipeline(ch0, nch):
            nwin = nch // _WCH
            dst_cp = pltpu.async_copy(dst_hbm.at[pl.ds(ch0, nch)],
                                      dst_v.at[pl.ds(0, nch)], dsem)
            load_idx(ch0, 0, 0)
            load_idx(ch0, 1, 1)
            wait_idx(0)
            start_gather(0, 0, 0)
            start_gather(0, 1, 1)
            dst_cp.wait()

            @pl.loop(0, nwin, step=2)
            def _(w):
                for r in range(2 * _WCH):
                    b = r % _NBUF
                    wait_gather(r // _WCH, r % _WCH, b)
                    if r == 3:
                        @pl.when(w + 2 < nwin)
                        def _():
                            load_idx(ch0, w + 2, 0)
                    if r == 7:
                        @pl.when(w + 3 < nwin)
                        def _():
                            load_idx(ch0, w + 3, 1)
                    scat = pltpu.async_copy(
                        rows_v.at[b], acc.at[dst_v.at[w * _WCH + r]],
                        ssem[b], add=True)
                    scat.wait()
                    t = r + 2
                    if t < 2 * _WCH:
                        if t == _WCH:
                            wait_idx(1)
                        start_gather(t // _WCH, t % _WCH, b)
                    else:
                        @pl.when(w + 2 < nwin)
                        def _(r=r, b=b):
                            if r == 2 * _WCH - 2:
                                wait_idx(0)
                            start_gather(0, r - (2 * _WCH - 2), b)

        plsc.subcore_barrier()

        @pl.when(c == 0)
        def _():
            run_pipeline(s * _CPW_A, _CPW_A)

        @pl.when(c == 1)
        def _():
            run_pipeline(_NS * _CPW_A + s * _CPW_B, _CPW_B)

        plsc.subcore_barrier()

        @pl.when(c == 0)
        def _():
            copy_slab(acc, o0)

        @pl.when(c == 1)
        def _():
            copy_slab(acc, o1)

    return k(h_pad, src2, dst2)


def _tc_linear(h, p0, p1, wt, bias, relu, pad_out):
    rows = _HROWS if pad_out else _N

    def body(h_ref, a_ref, b_ref, w_ref, bias_ref, o_ref):
        z = h_ref[pl.ds(0, _N), :] + a_ref[...] + b_ref[...]
        y = jnp.dot(z, w_ref[...], preferred_element_type=jnp.float32)
        y = y + bias_ref[...]
        o_ref[pl.ds(0, _N), :] = jnp.maximum(y, 0.0) if relu else y
        if pad_out:
            o_ref[pl.ds(_N, _HROWS - _N), :] = jnp.zeros(
                (_HROWS - _N, _F), jnp.float32)

    return pl.pallas_call(
        body,
        out_shape=jax.ShapeDtypeStruct((rows, _F), jnp.float32),
    )(h, p0, p1, wt, bias)


def kernel(x, adj, W1, b1, W2, b2):
    src = adj[0]
    dst = adj[1]
    pad = _EPAD - _E
    src2 = jnp.concatenate([src, jnp.full((pad,), _N, jnp.int32)]
                           ).reshape(_NW * _CPW, _CH)
    dst2 = jnp.concatenate([dst, jnp.zeros((pad,), jnp.int32)]
                           ).reshape(_NW * _CPW, _CH)
    tail = jnp.zeros((_HROWS - _N, _F), jnp.float32)
    x_pad = jnp.concatenate([x, tail])

    p0, p1 = _sc_aggregate(x_pad, src2, dst2)
    h_pad = _tc_linear(x_pad, p0, p1, W1.T, b1.reshape(1, _F), True, True)
    q0, q1 = _sc_aggregate(h_pad, src2, dst2)
    return _tc_linear(h_pad, q0, q1, W2.T, b2.reshape(1, _F), False, False)


# speedup vs baseline: 1.0044x; 1.0044x over previous
import functools

import jax
import jax.numpy as jnp
from jax import lax
from jax.experimental import pallas as pl
from jax.experimental.pallas import tpu as pltpu
from jax.experimental.pallas import tpu_sc as plsc

_N = 10000
_F = 128
_E = 320000
_NC = 2
_NS = 16
_NW = _NC * _NS
_CH = 128
_CPW = 80
_EPAD = _NW * _CPW * _CH
_RPS = 640
_NBUF = 2
_WCH = 4
_HROWS = _N + 8
_CPW_A = 128
_CPW_B = 32


def _sc_aggregate(h_pad, src2, dst2):
    mesh = plsc.VectorSubcoreMesh(core_axis_name="c", subcore_axis_name="s")
    out_t = (jax.ShapeDtypeStruct((_N, _F), jnp.float32),
             jax.ShapeDtypeStruct((_N, _F), jnp.float32))

    @functools.partial(
        pl.kernel, mesh=mesh, out_type=out_t,
        scratch_types=[
            pltpu.VMEM((2, _WCH, _CH), jnp.int32),
            pltpu.VMEM((max(_CPW_A, _CPW_B), _CH), jnp.int32),
            pltpu.VMEM((_NBUF, _CH, _F), jnp.float32),
            pltpu.VMEM_SHARED((_N, _F), jnp.float32),
        ] + [pltpu.SemaphoreType.DMA] * 7)
    def k(h_hbm, src_hbm, dst_hbm, o0, o1,
          src_v, dst_v, rows_v, acc, *sems):
        gsem = sems[0:2]
        ssem = sems[2:4]
        isem = sems[4:6]
        dsem = sems[6]
        c = lax.axis_index("c")
        s = lax.axis_index("s")
        base = s * _RPS

        zvec = jnp.zeros((16,), jnp.float32)

        @pl.loop(0, _CH)
        def _(i):
            @pl.loop(0, _F, step=16)
            def _(j):
                rows_v[0, i, pl.ds(j, 16)] = zvec

        @pl.loop(0, 3)
        def _(kk):
            pltpu.sync_copy(rows_v.at[0],
                            acc.at[pl.ds(base + kk * _CH, _CH)])

        @pl.when(s < _NS - 1)
        def _():
            pltpu.sync_copy(rows_v.at[0],
                            acc.at[pl.ds(base + 3 * _CH, _CH)])
            pltpu.sync_copy(rows_v.at[0],
                            acc.at[pl.ds(base + 4 * _CH, _CH)])

        @pl.when(s == _NS - 1)
        def _():
            pltpu.sync_copy(rows_v.at[0, pl.ds(0, 16)],
                            acc.at[pl.ds(base + 3 * _CH, 16)])

        def copy_slab(src_ref, dst_ref):
            @pl.when(s < _NS - 1)
            def _():
                pltpu.sync_copy(src_ref.at[pl.ds(base, _RPS)],
                                dst_ref.at[pl.ds(base, _RPS)])

            @pl.when(s == _NS - 1)
            def _():
                pltpu.sync_copy(src_ref.at[pl.ds(_RPS * (_NS - 1), 400)],
                                dst_ref.at[pl.ds(_RPS * (_NS - 1), 400)])

        def load_idx(ch0, win, p):
            pltpu.async_copy(
                src_hbm.at[pl.ds(ch0 + win * _WCH, _WCH)],
                src_v.at[p], isem[p])

        def wait_idx(p):
            pltpu.make_async_copy(
                src_hbm.at[pl.ds(0, _WCH)], src_v.at[p],
                isem[p]).wait()

        def start_gather(win_p, row, b):
            pltpu.async_copy(h_hbm.at[src_v.at[win_p, row]],
                             rows_v.at[b], gsem[b], priority=1)

        def wait_gather(win_p, row, b):
            pltpu.make_async_copy(h_hbm.at[src_v.at[win_p, row]],
                                  rows_v.at[b], gsem[b]).wait()

        def run_pipeline(ch0, nch):
            nwin = nch // _WCH
            dst_cp = pltpu.async_copy(dst_hbm.at[pl.ds(ch0, nch)],
                                      dst_v.at[pl.ds(0, nch)], dsem)
            load_idx(ch0, 0, 0)
            load_idx(ch0, 1, 1)
            wait_idx(0)
            start_gather(0, 0, 0)
            start_gather(0, 1, 1)
            dst_cp.wait()

            @pl.loop(0, nwin, step=2)
            def _(w):
                for r in range(2 * _WCH):
                    b = r % _NBUF
                    wait_gather(r // _WCH, r % _WCH, b)
                    if r == 3:
                        @pl.when(w + 2 < nwin)
                        def _():
                            load_idx(ch0, w + 2, 0)
                    if r == 7:
                        @pl.when(w + 3 < nwin)
                        def _():
                            load_idx(ch0, w + 3, 1)
                    scat = pltpu.async_copy(
                        rows_v.at[b], acc.at[dst_v.at[w * _WCH + r]],
                        ssem[b], add=True)
                    scat.wait()
                    t = r + 2
                    if t < 2 * _WCH:
                        if t == _WCH:
                            wait_idx(1)
                        start_gather(t // _WCH, t % _WCH, b)
                    else:
                        @pl.when(w + 2 < nwin)
                        def _(r=r, b=b):
                            if r == 2 * _WCH - 2:
                                wait_idx(0)
                            start_gather(0, r - (2 * _WCH - 2), b)

        plsc.subcore_barrier()

        @pl.when(c == 0)
        def _():
            run_pipeline(s * _CPW_A, _CPW_A)

        @pl.when(c == 1)
        def _():
            run_pipeline(_NS * _CPW_A + s * _CPW_B, _CPW_B)

        plsc.subcore_barrier()

        @pl.when(c == 0)
        def _():
            copy_slab(acc, o0)

        @pl.when(c == 1)
        def _():
            copy_slab(acc, o1)

    return k(h_pad, src2, dst2)


def _tc_linear(h, p0, p1, wt, bias, relu, pad_out):
    rows = _HROWS if pad_out else _N

    def body(h_ref, a_ref, b_ref, w_ref, bias_ref, o_ref):
        z = h_ref[pl.ds(0, _N), :] + a_ref[...] + b_ref[...]
        y = jnp.dot(z, w_ref[...], preferred_element_type=jnp.float32)
        y = y + bias_ref[...]
        o_ref[pl.ds(0, _N), :] = jnp.maximum(y, 0.0) if relu else y
        if pad_out:
            o_ref[pl.ds(_N, _HROWS - _N), :] = jnp.zeros(
                (_HROWS - _N, _F), jnp.float32)

    return pl.pallas_call(
        body,
        out_shape=jax.ShapeDtypeStruct((rows, _F), jnp.float32),
    )(h, p0, p1, wt, bias)


def kernel(x, adj, W1, b1, W2, b2):
    src = adj[0]
    dst = adj[1]
    pad = _EPAD - _E
    src2 = jnp.concatenate([src, jnp.full((pad,), _N, jnp.int32)]
                           ).reshape(_NW * _CPW, _CH)
    dst2 = jnp.concatenate([dst, jnp.zeros((pad,), jnp.int32)]
                           ).reshape(_NW * _CPW, _CH)
    tail = jnp.zeros((_HROWS - _N, _F), jnp.float32)
    x_pad = jnp.concatenate([x, tail])

    p0, p1 = _sc_aggregate(x_pad, src2, dst2)
    h_pad = _tc_linear(x_pad, p0, p1, W1.T, b1.reshape(1, _F), True, True)
    q0, q1 = _sc_aggregate(h_pad, src2, dst2)
    return _tc_linear(h_pad, q0, q1, W2.T, b2.reshape(1, _F), False, False)


# speedup vs baseline: 1.0054x; 1.0010x over previous
import functools

import jax
import jax.numpy as jnp
from jax import lax
from jax.experimental import pallas as pl
from jax.experimental.pallas import tpu as pltpu
from jax.experimental.pallas import tpu_sc as plsc

_N = 10000
_F = 128
_E = 320000
_NC = 2
_NS = 16
_NW = _NC * _NS
_CH = 128
_CPW = 80
_EPAD = _NW * _CPW * _CH
_RPS = 640
_NBUF = 2
_WCH = 4
_HROWS = _N + 8
_CPW_A = 128
_CPW_B = 32


def _sc_aggregate(h_pad, src2, dst2):
    mesh = plsc.VectorSubcoreMesh(core_axis_name="c", subcore_axis_name="s")
    out_t = (jax.ShapeDtypeStruct((_N, _F), jnp.float32),
             jax.ShapeDtypeStruct((_N, _F), jnp.float32))

    @functools.partial(
        pl.kernel, mesh=mesh, out_type=out_t,
        scratch_types=[
            pltpu.VMEM((2, _WCH, _CH), jnp.int32),
            pltpu.VMEM((max(_CPW_A, _CPW_B), _CH), jnp.int32),
            pltpu.VMEM((_NBUF, _CH, _F), jnp.float32),
            pltpu.VMEM_SHARED((_N, _F), jnp.float32),
        ] + [pltpu.SemaphoreType.DMA] * 7)
    def k(h_hbm, src_hbm, dst_hbm, o0, o1,
          src_v, dst_v, rows_v, acc, *sems):
        gsem = sems[0:2]
        ssem = sems[2:4]
        isem = sems[4:6]
        dsem = sems[6]
        c = lax.axis_index("c")
        s = lax.axis_index("s")
        base = s * _RPS

        zvec = jnp.zeros((16,), jnp.float32)

        @pl.loop(0, _CH)
        def _(i):
            @pl.loop(0, _F, step=16)
            def _(j):
                rows_v[0, i, pl.ds(j, 16)] = zvec

        @pl.loop(0, 3)
        def _(kk):
            pltpu.sync_copy(rows_v.at[0],
                            acc.at[pl.ds(base + kk * _CH, _CH)])

        @pl.when(s < _NS - 1)
        def _():
            pltpu.sync_copy(rows_v.at[0],
                            acc.at[pl.ds(base + 3 * _CH, _CH)])
            pltpu.sync_copy(rows_v.at[0],
                            acc.at[pl.ds(base + 4 * _CH, _CH)])

        @pl.when(s == _NS - 1)
        def _():
            pltpu.sync_copy(rows_v.at[0, pl.ds(0, 16)],
                            acc.at[pl.ds(base + 3 * _CH, 16)])

        def copy_slab(src_ref, dst_ref):
            @pl.when(s < _NS - 1)
            def _():
                pltpu.sync_copy(src_ref.at[pl.ds(base, _RPS)],
                                dst_ref.at[pl.ds(base, _RPS)])

            @pl.when(s == _NS - 1)
            def _():
                pltpu.sync_copy(src_ref.at[pl.ds(_RPS * (_NS - 1), 400)],
                                dst_ref.at[pl.ds(_RPS * (_NS - 1), 400)])

        def load_idx(ch0, win, p):
            pltpu.async_copy(
                src_hbm.at[pl.ds(ch0 + win * _WCH, _WCH)],
                src_v.at[p], isem[p])

        def wait_idx(p):
            pltpu.make_async_copy(
                src_hbm.at[pl.ds(0, _WCH)], src_v.at[p],
                isem[p]).wait()

        def start_gather(win_p, row, b):
            pltpu.async_copy(h_hbm.at[src_v.at[win_p, row]],
                             rows_v.at[b], gsem[b])

        def wait_gather(win_p, row, b):
            pltpu.make_async_copy(h_hbm.at[src_v.at[win_p, row]],
                                  rows_v.at[b], gsem[b]).wait()

        def run_pipeline(ch0, nch):
            nwin = nch // _WCH
            dst_cp = pltpu.async_copy(dst_hbm.at[pl.ds(ch0, nch)],
                                      dst_v.at[pl.ds(0, nch)], dsem)
            load_idx(ch0, 0, 0)
            load_idx(ch0, 1, 1)
            wait_idx(0)
            start_gather(0, 0, 0)
            start_gather(0, 1, 1)
            dst_cp.wait()

            @pl.loop(0, nwin, step=2)
            def _(w):
                for r in range(2 * _WCH):
                    b = r % _NBUF
                    wait_gather(r // _WCH, r % _WCH, b)
                    if r == 3:
                        @pl.when(w + 2 < nwin)
                        def _():
                            load_idx(ch0, w + 2, 0)
                    if r == 7:
                        @pl.when(w + 3 < nwin)
                        def _():
                            load_idx(ch0, w + 3, 1)
                    scat = pltpu.async_copy(
                        rows_v.at[b], acc.at[dst_v.at[w * _WCH + r]],
                        ssem[b], add=True)
                    scat.wait()
                    t = r + 2
                    if t < 2 * _WCH:
                        if t == _WCH:
                            wait_idx(1)
                        start_gather(t // _WCH, t % _WCH, b)
                    else:
                        @pl.when(w + 2 < nwin)
                        def _(r=r, b=b):
                            if r == 2 * _WCH - 2:
                                wait_idx(0)
                            start_gather(0, r - (2 * _WCH - 2), b)

        plsc.subcore_barrier()

        @pl.when(c == 0)
        def _():
            run_pipeline(s * _CPW_A, _CPW_A)

        @pl.when(c == 1)
        def _():
            run_pipeline(_NS * _CPW_A + s * _CPW_B, _CPW_B)

        plsc.subcore_barrier()

        @pl.when(c == 0)
        def _():
            copy_slab(acc, o0)

        @pl.when(c == 1)
        def _():
            copy_slab(acc, o1)

    return k(h_pad, src2, dst2)


def _tc_linear(h, p0, p1, wt, bias, relu, pad_out):
    rows = _HROWS if pad_out else _N

    def body(h_ref, a_ref, b_ref, w_ref, bias_ref, o_ref):
        z = h_ref[pl.ds(0, _N), :] + a_ref[...] + b_ref[...]
        y = jnp.dot(z, w_ref[...], preferred_element_type=jnp.float32)
        y = y + bias_ref[...]
        o_ref[pl.ds(0, _N), :] = jnp.maximum(y, 0.0) if relu else y
        if pad_out:
            o_ref[pl.ds(_N, _HROWS - _N), :] = jnp.zeros(
                (_HROWS - _N, _F), jnp.float32)

    return pl.pallas_call(
        body,
        out_shape=jax.ShapeDtypeStruct((rows, _F), jnp.float32),
    )(h, p0, p1, wt, bias)


def kernel(x, adj, W1, b1, W2, b2):
    src = adj[0]
    dst = adj[1]
    pad = _EPAD - _E
    src2 = jnp.concatenate([src, jnp.full((pad,), _N, jnp.int32)]
                           ).reshape(_NW * _CPW, _CH)
    dst2 = jnp.concatenate([dst, jnp.zeros((pad,), jnp.int32)]
                           ).reshape(_NW * _CPW, _CH)
    tail = jnp.zeros((_HROWS - _N, _F), jnp.float32)
    x_pad = jnp.concatenate([x, tail])

    p0, p1 = _sc_aggregate(x_pad, src2, dst2)
    h_pad = _tc_linear(x_pad, p0, p1, W1.T, b1.reshape(1, _F), True, True)
    q0, q1 = _sc_aggregate(h_pad, src2, dst2)
    return _tc_linear(h_pad, q0, q1, W2.T, b2.reshape(1, _F), False, False)
